# baseline (speedup 1.0000x reference)
import functools

import jax
import jax.numpy as jnp
from jax import lax
from jax.experimental import pallas as pl
from jax.experimental.pallas import tpu as pltpu
from jax.experimental.pallas import tpu_sc as plsc

B = 4096
L = 50
D = 64
NC = 2
NS = 16
NW = NC * NS
QPW = B // NW
QPC = 8
RPC = QPC * L
NCHUNK = QPW // QPC
NLANE = 16
NDV = D // NLANE
NBUF = 4


def _bag_kernel(q_hbm, t_hbm, out_hbm, idx_v, bufs, out_v,
                sem0, sem1, sem2, sem3):
    wid = lax.axis_index("c") * NS + lax.axis_index("s")
    pltpu.sync_copy(q_hbm.at[wid], idx_v)

    sems = (sem0, sem1, sem2, sem3)
    for b in range(NBUF):
        pltpu.make_async_copy(
            t_hbm.at[idx_v.at[b]], bufs.at[b], sems[b]).start()

    inv = jnp.float32(1.0 / B)

    def outer(co, carry):
        for b in range(NBUF):
            c = co * NBUF + b
            pltpu.make_async_copy(
                t_hbm.at[idx_v.at[c]], bufs.at[b], sems[b]).wait()
            zeros = [jnp.zeros((NLANE,), jnp.float32)] * (QPC * NDV)

            @plsc.parallel_loop(0, L, unroll=5, carry=zeros)
            def accs(l, acc):
                new = []
                for q in range(QPC):
                    for d in range(NDV):
                        new.append(
                            acc[q * NDV + d]
                            + bufs[b, q * L + l, pl.ds(d * NLANE, NLANE)])
                return new

            for q in range(QPC):
                for d in range(NDV):
                    out_v[c * QPC + q, pl.ds(d * NLANE, NLANE)] = (
                        accs[q * NDV + d] * inv)
            @pl.when(c + NBUF < NCHUNK)
            def _():
                pltpu.make_async_copy(
                    t_hbm.at[idx_v.at[c + NBUF]], bufs.at[b],
                    sems[b]).start()
        return carry

    lax.fori_loop(0, NCHUNK // NBUF, outer, None)

    pltpu.sync_copy(out_v, out_hbm.at[pl.ds(wid * QPW, QPW)])


@jax.jit
def _run(q3, table):
    mesh = plsc.VectorSubcoreMesh(core_axis_name="c", subcore_axis_name="s")
    return pl.kernel(
        _bag_kernel,
        mesh=mesh,
        compiler_params=pltpu.CompilerParams(use_tc_tiling_on_sc=False),
        out_type=jax.ShapeDtypeStruct((B, D), jnp.float32),
        scratch_types=[
            pltpu.VMEM((NCHUNK, RPC), jnp.int32),
            pltpu.VMEM((NBUF, RPC, D), jnp.float32),
            pltpu.VMEM((QPW, D), jnp.float32),
            pltpu.SemaphoreType.DMA,
            pltpu.SemaphoreType.DMA,
            pltpu.SemaphoreType.DMA,
            pltpu.SemaphoreType.DMA,
        ],
    )(q3, table)


def kernel(query, table):
    q3 = query.reshape(NW, NCHUNK, RPC).astype(jnp.int32)
    return _run(q3, table)


# speedup vs baseline: 1.0038x; 1.0038x over previous
import functools

import jax
import jax.numpy as jnp
from jax import lax
from jax.experimental import pallas as pl
from jax.experimental.pallas import tpu as pltpu
from jax.experimental.pallas import tpu_sc as plsc

B = 4096
L = 50
D = 64
NC = 2
NS = 16
NW = NC * NS
QPW = B // NW
QPC = 4
RPC = QPC * L
NCHUNK = QPW // QPC
NLANE = 16
NDV = D // NLANE
NBUF = 8


def _bag_kernel(q_hbm, t_hbm, out_hbm, idx_v, bufs, out_v,
                sem0, sem1, sem2, sem3, sem4, sem5, sem6, sem7):
    wid = lax.axis_index("c") * NS + lax.axis_index("s")
    pltpu.sync_copy(q_hbm.at[wid], idx_v)

    sems = (sem0, sem1, sem2, sem3, sem4, sem5, sem6, sem7)
    for b in range(NBUF):
        pltpu.make_async_copy(
            t_hbm.at[idx_v.at[b]], bufs.at[b], sems[b]).start()

    inv = jnp.float32(1.0 / B)

    def outer(co, carry):
        for b in range(NBUF):
            c = co * NBUF + b
            pltpu.make_async_copy(
                t_hbm.at[idx_v.at[c]], bufs.at[b], sems[b]).wait()
            zeros = [jnp.zeros((NLANE,), jnp.float32)] * (QPC * NDV)

            @plsc.parallel_loop(0, L, unroll=5, carry=zeros)
            def accs(l, acc):
                new = []
                for q in range(QPC):
                    for d in range(NDV):
                        new.append(
                            acc[q * NDV + d]
                            + bufs[b, q * L + l, pl.ds(d * NLANE, NLANE)])
                return new

            for q in range(QPC):
                for d in range(NDV):
                    out_v[c * QPC + q, pl.ds(d * NLANE, NLANE)] = (
                        accs[q * NDV + d] * inv)
            @pl.when(c + NBUF < NCHUNK)
            def _():
                pltpu.make_async_copy(
                    t_hbm.at[idx_v.at[c + NBUF]], bufs.at[b],
                    sems[b]).start()
        return carry

    lax.fori_loop(0, NCHUNK // NBUF, outer, None)

    pltpu.sync_copy(out_v, out_hbm.at[pl.ds(wid * QPW, QPW)])


@jax.jit
def _run(q3, table):
    mesh = plsc.VectorSubcoreMesh(core_axis_name="c", subcore_axis_name="s")
    return pl.kernel(
        _bag_kernel,
        mesh=mesh,
        compiler_params=pltpu.CompilerParams(use_tc_tiling_on_sc=False),
        out_type=jax.ShapeDtypeStruct((B, D), jnp.float32),
        scratch_types=[
            pltpu.VMEM((NCHUNK, RPC), jnp.int32),
            pltpu.VMEM((NBUF, RPC, D), jnp.float32),
            pltpu.VMEM((QPW, D), jnp.float32),
            pltpu.SemaphoreType.DMA,
            pltpu.SemaphoreType.DMA,
            pltpu.SemaphoreType.DMA,
            pltpu.SemaphoreType.DMA,
            pltpu.SemaphoreType.DMA,
            pltpu.SemaphoreType.DMA,
            pltpu.SemaphoreType.DMA,
            pltpu.SemaphoreType.DMA,
        ],
    )(q3, table)


def kernel(query, table):
    q3 = query.reshape(NW, NCHUNK, RPC).astype(jnp.int32)
    return _run(q3, table)


# speedup vs baseline: 1.0126x; 1.0088x over previous
import functools

import jax
import jax.numpy as jnp
from jax import lax
from jax.experimental import pallas as pl
from jax.experimental.pallas import tpu as pltpu
from jax.experimental.pallas import tpu_sc as plsc

B = 4096
L = 50
D = 64
NC = 2
NS = 16
NW = NC * NS
QPW = B // NW
QPC = 4
RPC = QPC * L
NCHUNK = QPW // QPC
NLANE = 16
NDV = D // NLANE
NBUF = 4


def _bag_kernel(q_hbm, t_hbm, out_hbm, idx_v, bufs, out_v,
                sem0, sem1, sem2, sem3):
    wid = lax.axis_index("c") * NS + lax.axis_index("s")
    pltpu.sync_copy(q_hbm.at[wid], idx_v)

    sems = (sem0, sem1, sem2, sem3)
    for b in range(NBUF):
        pltpu.make_async_copy(
            t_hbm.at[idx_v.at[b]], bufs.at[b], sems[b]).start()

    inv = jnp.float32(1.0 / B)

    def outer(co, carry):
        for b in range(NBUF):
            c = co * NBUF + b
            pltpu.make_async_copy(
                t_hbm.at[idx_v.at[c]], bufs.at[b], sems[b]).wait()
            zeros = [jnp.zeros((NLANE,), jnp.float32)] * (QPC * NDV)

            @plsc.parallel_loop(0, L, unroll=5, carry=zeros)
            def accs(l, acc):
                new = []
                for q in range(QPC):
                    for d in range(NDV):
                        new.append(
                            acc[q * NDV + d]
                            + bufs[b, q * L + l, pl.ds(d * NLANE, NLANE)])
                return new

            for q in range(QPC):
                for d in range(NDV):
                    out_v[c * QPC + q, pl.ds(d * NLANE, NLANE)] = (
                        accs[q * NDV + d] * inv)
            @pl.when(c + NBUF < NCHUNK)
            def _():
                pltpu.make_async_copy(
                    t_hbm.at[idx_v.at[c + NBUF]], bufs.at[b],
                    sems[b]).start()
        return carry

    lax.fori_loop(0, NCHUNK // NBUF, outer, None)

    pltpu.sync_copy(out_v, out_hbm.at[pl.ds(wid * QPW, QPW)])


@jax.jit
def _run(q3, table):
    mesh = plsc.VectorSubcoreMesh(core_axis_name="c", subcore_axis_name="s")
    return pl.kernel(
        _bag_kernel,
        mesh=mesh,
        compiler_params=pltpu.CompilerParams(
            use_tc_tiling_on_sc=False,
            disable_bounds_checks=True,
            disable_semaphore_checks=True,
        ),
        out_type=jax.ShapeDtypeStruct((B, D), jnp.float32),
        scratch_types=[
            pltpu.VMEM((NCHUNK, RPC), jnp.int32),
            pltpu.VMEM((NBUF, RPC, D), jnp.float32),
            pltpu.VMEM((QPW, D), jnp.float32),
            pltpu.SemaphoreType.DMA,
            pltpu.SemaphoreType.DMA,
            pltpu.SemaphoreType.DMA,
            pltpu.SemaphoreType.DMA,
        ],
    )(q3, table)


def kernel(query, table):
    q3 = query.reshape(NW, NCHUNK, RPC).astype(jnp.int32)
    return _run(q3, table)


# speedup vs baseline: 1.0134x; 1.0008x over previous
import jax
import jax.numpy as jnp
from jax import lax
from jax.experimental import pallas as pl
from jax.experimental.pallas import tpu as pltpu
from jax.experimental.pallas import tpu_sc as plsc

B = 4096
L = 50
D = 64
NC = 2
NS = 16
NW = NC * NS
QPW = B // NW
QPC = 4
RPC = QPC * L
NCHUNK = QPW // QPC
NLANE = 16
NDV = D // NLANE
NBUF = 4


def _bag_kernel(q_hbm, t_hbm, out_hbm, idx_v, bufs, out_v,
                sem0, sem1, sem2, sem3):
    wid = lax.axis_index("c") * NS + lax.axis_index("s")
    pltpu.sync_copy(q_hbm.at[wid], idx_v)

    sems = (sem0, sem1, sem2, sem3)
    for b in range(NBUF):
        pltpu.make_async_copy(
            t_hbm.at[idx_v.at[b]], bufs.at[b], sems[b]).start()

    inv = jnp.float32(1.0 / B)

    def outer(co, carry):
        for b in range(NBUF):
            c = co * NBUF + b
            pltpu.make_async_copy(
                t_hbm.at[idx_v.at[c]], bufs.at[b], sems[b]).wait()
            zeros = [jnp.zeros((NLANE,), jnp.float32)] * (QPC * NDV)

            @plsc.parallel_loop(0, L, unroll=5, carry=zeros)
            def accs(l, acc):
                new = []
                for q in range(QPC):
                    for d in range(NDV):
                        new.append(
                            acc[q * NDV + d]
                            + bufs[b, q * L + l, pl.ds(d * NLANE, NLANE)])
                return new

            for q in range(QPC):
                for d in range(NDV):
                    out_v[c * QPC + q, pl.ds(d * NLANE, NLANE)] = (
                        accs[q * NDV + d] * inv)
            @pl.when(c + NBUF < NCHUNK)
            def _():
                pltpu.make_async_copy(
                    t_hbm.at[idx_v.at[c + NBUF]], bufs.at[b],
                    sems[b]).start()
        return carry

    lax.fori_loop(0, NCHUNK // NBUF, outer, None)

    pltpu.sync_copy(out_v, out_hbm.at[pl.ds(wid * QPW, QPW)])


@jax.jit
def _run(q3, table):
    mesh = plsc.VectorSubcoreMesh(core_axis_name="c", subcore_axis_name="s")
    return pl.kernel(
        _bag_kernel,
        mesh=mesh,
        compiler_params=pltpu.CompilerParams(use_tc_tiling_on_sc=False),
        out_type=jax.ShapeDtypeStruct((B, D), jnp.float32),
        scratch_types=[
            pltpu.VMEM((NCHUNK, RPC), jnp.int32),
            pltpu.VMEM((NBUF, RPC, D), jnp.float32),
            pltpu.VMEM((QPW, D), jnp.float32),
            pltpu.SemaphoreType.DMA,
            pltpu.SemaphoreType.DMA,
            pltpu.SemaphoreType.DMA,
            pltpu.SemaphoreType.DMA,
        ],
    )(q3, table)


def kernel(query, table):
    q3 = query.reshape(NW, NCHUNK, RPC).astype(jnp.int32)
    return _run(q3, table)
